# baseline (speedup 1.0000x reference)
import functools

import jax
import jax.numpy as jnp
from jax.experimental import pallas as pl
from jax.experimental.pallas import tpu as pltpu
from jax.experimental.pallas import tpu_sc as plsc

H = 2048
I = 1024
E = 16
G = 4
GS = 4
SH_I = 2048
T = 2048
S = 2048
MR = 256
NMR = T // MR
MB = 256
NB = 2 * T // MB + 4
NPAD = NB * MB
NC = 2
NS = 16
NW = NC * NS
BPW = T // NW
MS = 256
BF = jnp.bfloat16
F32 = jnp.float32
I32 = jnp.int32
VMEM_LIMIT = 64 * 1024 * 1024


def _dot_t(a, b):
    return jax.lax.dot_general(a, b, (((1,), (1,)), ((), ())),
                               preferred_element_type=F32)


def _route_kernel(hs_ref, gw_ref, meta_ref, pos_ref, grp_ref,
                  cnt_ref, a_ref, rank_ref):
    p = pl.program_id(0)
    m = pl.program_id(1)

    @pl.when((p == 0) & (m == 0))
    def _():
        cnt_ref[...] = jnp.zeros((8, 128), F32)

    @pl.when(p == 0)
    def _():
        x = hs_ref[...]
        logits = _dot_t(x, gw_ref[...])
        mx = jnp.max(logits, axis=1, keepdims=True)
        ex = jnp.exp(logits - mx)
        prob = ex / jnp.sum(ex, axis=1, keepdims=True)
        iota = jax.lax.broadcasted_iota(I32, (MR, E), 1)
        w1 = jnp.max(prob, axis=1, keepdims=True)
        i1 = jnp.min(jnp.where(prob == w1, iota, E), axis=1, keepdims=True)
        p2 = jnp.where(iota == i1, -1.0, prob)
        w2 = jnp.max(p2, axis=1, keepdims=True)
        i2 = jnp.min(jnp.where(p2 == w2, iota, E), axis=1, keepdims=True)
        g0 = i1 // GS
        s0 = i1 - g0 * GS
        g1 = i2 // GS
        s1 = i2 - g1 * GS
        same = (g0 == g1).astype(F32)
        iota4 = jax.lax.broadcasted_iota(I32, (MR, GS), 1)
        oh0 = (iota4 == s0).astype(F32)
        oh1 = (iota4 == s1).astype(F32)
        flat0 = w1 * oh0 + same * w2 * oh1
        flat1 = w2 * oh1
        scal0 = w1 + same * w2
        scal1 = (1.0 - same) * w2
        zeros123 = jnp.zeros((MR, 123), F32)
        for k, (flat, scal) in enumerate(((flat0, scal0), (flat1, scal1))):
            fm = jnp.where(flat == 0.0, -1e9, flat)
            mxg = jnp.max(fm, axis=1, keepdims=True)
            eg = jnp.exp(fm - mxg)
            sm = eg / jnp.sum(eg, axis=1, keepdims=True)
            meta_ref[k, pl.ds(m * MR, MR), :] = jnp.concatenate(
                [sm, scal, zeros123], axis=1)
        a_ref[0, pl.ds(m * MR, MR), :] = (iota4 == g0).astype(F32)
        a_ref[1, pl.ds(m * MR, MR), :] = (iota4 == g1).astype(F32)

    @pl.when(p < 2)
    def _():
        a = a_ref[p, pl.ds(m * MR, MR), :]
        ri = jax.lax.broadcasted_iota(I32, (MR, MR), 0)
        ci = jax.lax.broadcasted_iota(I32, (MR, MR), 1)
        tril = (ri > ci).astype(BF)
        r_intra = jax.lax.dot_general(
            tril, a.astype(BF), (((1,), (0,)), ((), ())),
            preferred_element_type=F32)
        base = cnt_ref[0:1, 0:4]
        rank = jnp.sum(a * (r_intra + base), axis=1, keepdims=True)
        rank_ref[p, pl.ds(m * MR, MR), :] = rank
        cnt_ref[0:1, 0:4] = base + jnp.sum(a, axis=0, keepdims=True)

    @pl.when(p == 2)
    def _():
        cnt = cnt_ref[0:1, 0:4]
        pad = jnp.ceil(cnt / MB) * MB
        b1 = pad[0:1, 0:1]
        b2 = b1 + pad[0:1, 1:2]
        b3 = b2 + pad[0:1, 2:3]
        for k in range(2):
            a = a_ref[k, pl.ds(m * MR, MR), :]
            r = rank_ref[k, pl.ds(m * MR, MR), :]
            posk = (r + a[:, 1:2] * b1 + a[:, 2:3] * b2 + a[:, 3:4] * b3)
            pos_ref[pl.ds(m * MR, MR), k:k + 1] = posk.astype(I32)

        @pl.when(m == 0)
        def _():
            iob = jax.lax.broadcasted_iota(I32, (8, 128), 1).astype(F32) * MB
            grp = ((iob >= b1).astype(I32) + (iob >= b2).astype(I32)
                   + (iob >= b3).astype(I32))
            grp_ref[...] = grp


def _route(hs, gate_w):
    return pl.pallas_call(
        _route_kernel,
        grid=(3, NMR),
        in_specs=[
            pl.BlockSpec((MR, H), lambda p, m: (m, 0)),
            pl.BlockSpec((E, H), lambda p, m: (0, 0)),
        ],
        out_specs=[
            pl.BlockSpec((2, T, 128), lambda p, m: (0, 0, 0)),
            pl.BlockSpec((T, 8), lambda p, m: (0, 0)),
            pl.BlockSpec((8, 128), lambda p, m: (0, 0)),
        ],
        out_shape=[
            jax.ShapeDtypeStruct((2, T, 128), F32),
            jax.ShapeDtypeStruct((T, 8), I32),
            jax.ShapeDtypeStruct((8, 128), I32),
        ],
        scratch_shapes=[
            pltpu.VMEM((8, 128), F32),
            pltpu.VMEM((2, T, GS), F32),
            pltpu.VMEM((2, T, 1), F32),
        ],
        compiler_params=pltpu.CompilerParams(vmem_limit_bytes=VMEM_LIMIT),
    )(hs, gate_w)


def _dispatch_sc(hsb3, meta, pos_c):
    mesh = plsc.VectorSubcoreMesh(core_axis_name="c", subcore_axis_name="s")

    @functools.partial(
        pl.kernel, mesh=mesh,
        out_type=[jax.ShapeDtypeStruct((NPAD, 8, 128), I32),
                  jax.ShapeDtypeStruct((NPAD, 128), F32)],
        scratch_types=[
            pltpu.VMEM((2, BPW), I32),
            pltpu.VMEM((BPW, 8, 128), I32),
            pltpu.VMEM((2, BPW, 128), F32),
            pltpu.SemaphoreType.DMA,
        ])
    def disp(hsb_hbm, meta_hbm, pos_hbm, xg_hbm, ms_hbm,
             idx_v, rows_v, mrow_v, sem):
        wid = jax.lax.axis_index("s") * NC + jax.lax.axis_index("c")
        base = wid * BPW
        pltpu.sync_copy(pos_hbm.at[0, pl.ds(base, BPW)], idx_v.at[0])
        pltpu.sync_copy(pos_hbm.at[1, pl.ds(base, BPW)], idx_v.at[1])
        pltpu.sync_copy(hsb_hbm.at[pl.ds(base, BPW)], rows_v)
        pltpu.sync_copy(meta_hbm.at[0, pl.ds(base, BPW)], mrow_v.at[0])
        pltpu.sync_copy(meta_hbm.at[1, pl.ds(base, BPW)], mrow_v.at[1])
        c0 = pltpu.async_copy(rows_v, xg_hbm.at[idx_v.at[0]], sem)
        c1 = pltpu.async_copy(rows_v, xg_hbm.at[idx_v.at[1]], sem)
        c2 = pltpu.async_copy(mrow_v.at[0], ms_hbm.at[idx_v.at[0]], sem)
        c3 = pltpu.async_copy(mrow_v.at[1], ms_hbm.at[idx_v.at[1]], sem)
        c0.wait()
        c1.wait()
        c2.wait()
        c3.wait()

    return disp(hsb3, meta, pos_c)


def _moe_kernel(grp_ref, xg_ref, meta_ref, wg_ref, wu_ref, wd_ref, wm_ref,
                og_ref):
    fsm = meta_ref[:, 0:4]
    scal = meta_ref[:, 4:5]
    delta = _dot_t(fsm.astype(BF), wm_ref[0].astype(BF))
    xb = (xg_ref[...].astype(F32) + delta).astype(BF)
    gate = _dot_t(xb, wg_ref[0])
    up = _dot_t(xb, wu_ref[0])
    hb = (gate * jax.nn.sigmoid(gate) * up).astype(BF)
    og = _dot_t(hb, wd_ref[0])
    og_ref[...] = (scal * og).astype(BF)


def _moe(grp, xg, meta_s, wg, wu, wd, Wm):
    grid_spec = pltpu.PrefetchScalarGridSpec(
        num_scalar_prefetch=1,
        grid=(NB,),
        in_specs=[
            pl.BlockSpec((MB, H), lambda b, g: (b, 0)),
            pl.BlockSpec((MB, 128), lambda b, g: (b, 0)),
            pl.BlockSpec((1, I, H), lambda b, g: (g[b], 0, 0)),
            pl.BlockSpec((1, I, H), lambda b, g: (g[b], 0, 0)),
            pl.BlockSpec((1, H, I), lambda b, g: (g[b], 0, 0)),
            pl.BlockSpec((1, H, GS), lambda b, g: (g[b], 0, 0)),
        ],
        out_specs=pl.BlockSpec((MB, H), lambda b, g: (b, 0)),
    )
    return pl.pallas_call(
        _moe_kernel,
        grid_spec=grid_spec,
        out_shape=jax.ShapeDtypeStruct((NPAD, H), BF),
        compiler_params=pltpu.CompilerParams(vmem_limit_bytes=VMEM_LIMIT),
    )(grp, xg, meta_s, wg, wu, wd, Wm)


def _combine_sc(og3, pos_c):
    mesh = plsc.VectorSubcoreMesh(core_axis_name="c", subcore_axis_name="s")

    @functools.partial(
        pl.kernel, mesh=mesh,
        out_type=jax.ShapeDtypeStruct((2, T, 8, 128), I32),
        scratch_types=[
            pltpu.VMEM((2, BPW), I32),
            pltpu.VMEM((BPW, 8, 128), I32),
            pltpu.SemaphoreType.DMA,
        ])
    def comb(og_hbm, pos_hbm, y_hbm, idx_v, rows_v, sem):
        wid = jax.lax.axis_index("s") * NC + jax.lax.axis_index("c")
        base = wid * BPW
        pltpu.sync_copy(pos_hbm.at[0, pl.ds(base, BPW)], idx_v.at[0])
        pltpu.sync_copy(pos_hbm.at[1, pl.ds(base, BPW)], idx_v.at[1])
        pltpu.async_copy(og_hbm.at[idx_v.at[0]], rows_v, sem).wait()
        pltpu.sync_copy(rows_v, y_hbm.at[0, pl.ds(base, BPW)])
        pltpu.async_copy(og_hbm.at[idx_v.at[1]], rows_v, sem).wait()
        pltpu.sync_copy(rows_v, y_hbm.at[1, pl.ds(base, BPW)])

    return comb(og3, pos_c)


def _shared_kernel(hs_ref, wsg_ref, wsu_ref, wsd_ref, out_ref):
    xb = hs_ref[...].astype(BF)
    gate = _dot_t(xb, wsg_ref[...])
    up = _dot_t(xb, wsu_ref[...])
    hb = (gate * jax.nn.sigmoid(gate) * up).astype(BF)
    out_ref[...] = _dot_t(hb, wsd_ref[...])


def _shared(hs, wsg, wsu, wsd):
    return pl.pallas_call(
        _shared_kernel,
        grid=(T // MS,),
        in_specs=[
            pl.BlockSpec((MS, H), lambda m: (m, 0)),
            pl.BlockSpec((SH_I, H), lambda m: (0, 0)),
            pl.BlockSpec((SH_I, H), lambda m: (0, 0)),
            pl.BlockSpec((H, SH_I), lambda m: (0, 0)),
        ],
        out_specs=pl.BlockSpec((MS, H), lambda m: (m, 0)),
        out_shape=jax.ShapeDtypeStruct((T, H), F32),
        compiler_params=pltpu.CompilerParams(vmem_limit_bytes=VMEM_LIMIT),
    )(hs, wsg, wsu, wsd)


def _final_kernel(sh_ref, y_ref, out_ref):
    out_ref[...] = (y_ref[0].astype(F32) + y_ref[1].astype(F32) + sh_ref[...])


def _final(sh, y):
    return pl.pallas_call(
        _final_kernel,
        grid=(T // MS,),
        in_specs=[
            pl.BlockSpec((MS, H), lambda m: (m, 0)),
            pl.BlockSpec((2, MS, H), lambda m: (0, m, 0)),
        ],
        out_specs=pl.BlockSpec((MS, H), lambda m: (m, 0)),
        out_shape=jax.ShapeDtypeStruct((T, H), F32),
        compiler_params=pltpu.CompilerParams(vmem_limit_bytes=VMEM_LIMIT),
    )(sh, y)


def kernel(hidden_states, gate_w, Wg, Wu, Wd, Wm, Wsg, Wsu, Wsd):
    hs = hidden_states.reshape(T, H)
    wg = Wg.astype(BF)
    wu = Wu.astype(BF)
    wd = Wd.astype(BF)
    wsg = Wsg.astype(BF)
    wsu = Wsu.astype(BF)
    wsd = Wsd.astype(BF)
    hsb = hs.astype(BF)

    meta, pos, grp8 = _route(hs, gate_w)
    shared = _shared(hs, wsg, wsu, wsd)
    pos_c = pos.T[:2]
    grp = grp8[0, :NB]

    hsb_i = jax.lax.bitcast_convert_type(
        hsb.reshape(T, 8, 128, 2), I32)
    xg3, meta_s = _dispatch_sc(hsb_i, meta, pos_c)
    xg = jax.lax.bitcast_convert_type(xg3, BF).reshape(NPAD, H)

    og = _moe(grp, xg, meta_s, wg, wu, wd, Wm)

    og_i = jax.lax.bitcast_convert_type(
        og.reshape(NPAD, 8, 128, 2), I32)
    y4 = _combine_sc(og_i, pos_c)
    y = jax.lax.bitcast_convert_type(y4, BF).reshape(2, T, H)

    out = _final(shared, y)
    return out.reshape(1, S, H)


# speedup vs baseline: 2.8155x; 2.8155x over previous
import functools

import jax
import jax.numpy as jnp
from jax.experimental import pallas as pl
from jax.experimental.pallas import tpu as pltpu
from jax.experimental.pallas import tpu_sc as plsc

H = 2048
I = 1024
E = 16
G = 4
GS = 4
SH_I = 2048
T = 2048
S = 2048
MR = 256
NMR = T // MR
MB = 256
NB = 2 * T // MB + 4
NPAD = NB * MB
NC = 2
NS = 16
NW = NC * NS
BPW = T // NW
CH = 32
MS = 256
BF = jnp.bfloat16
F32 = jnp.float32
I32 = jnp.int32
VMEM_LIMIT = 64 * 1024 * 1024


def _dot_t(a, b):
    return jax.lax.dot_general(a, b, (((1,), (1,)), ((), ())),
                               preferred_element_type=F32)


def _route_kernel(hs_ref, gw_ref, meta_ref, pos_ref, grp_ref,
                  cnt_ref, a_ref, rank_ref):
    p = pl.program_id(0)
    m = pl.program_id(1)

    @pl.when((p == 0) & (m == 0))
    def _():
        cnt_ref[...] = jnp.zeros((8, 128), F32)

    @pl.when(p == 0)
    def _():
        x = hs_ref[...]
        logits = _dot_t(x, gw_ref[...])
        mx = jnp.max(logits, axis=1, keepdims=True)
        ex = jnp.exp(logits - mx)
        prob = ex / jnp.sum(ex, axis=1, keepdims=True)
        iota = jax.lax.broadcasted_iota(I32, (MR, E), 1)
        w1 = jnp.max(prob, axis=1, keepdims=True)
        i1 = jnp.min(jnp.where(prob == w1, iota, E), axis=1, keepdims=True)
        p2 = jnp.where(iota == i1, -1.0, prob)
        w2 = jnp.max(p2, axis=1, keepdims=True)
        i2 = jnp.min(jnp.where(p2 == w2, iota, E), axis=1, keepdims=True)
        g0 = i1 // GS
        s0 = i1 - g0 * GS
        g1 = i2 // GS
        s1 = i2 - g1 * GS
        same = (g0 == g1).astype(F32)
        iota4 = jax.lax.broadcasted_iota(I32, (MR, GS), 1)
        oh0 = (iota4 == s0).astype(F32)
        oh1 = (iota4 == s1).astype(F32)
        flat0 = w1 * oh0 + same * w2 * oh1
        flat1 = w2 * oh1
        scal0 = w1 + same * w2
        scal1 = (1.0 - same) * w2
        zeros123 = jnp.zeros((MR, 123), F32)
        for k, (flat, scal) in enumerate(((flat0, scal0), (flat1, scal1))):
            fm = jnp.where(flat == 0.0, -1e9, flat)
            mxg = jnp.max(fm, axis=1, keepdims=True)
            eg = jnp.exp(fm - mxg)
            sm = eg / jnp.sum(eg, axis=1, keepdims=True)
            meta_ref[k, pl.ds(m * MR, MR), :] = jnp.concatenate(
                [sm, scal, zeros123], axis=1)
        a_ref[0, pl.ds(m * MR, MR), :] = (iota4 == g0).astype(F32)
        a_ref[1, pl.ds(m * MR, MR), :] = (iota4 == g1).astype(F32)

    @pl.when(p < 2)
    def _():
        a = a_ref[p, pl.ds(m * MR, MR), :]
        ri = jax.lax.broadcasted_iota(I32, (MR, MR), 0)
        ci = jax.lax.broadcasted_iota(I32, (MR, MR), 1)
        tril = (ri > ci).astype(BF)
        r_intra = jax.lax.dot_general(
            tril, a.astype(BF), (((1,), (0,)), ((), ())),
            preferred_element_type=F32)
        base = cnt_ref[0:1, 0:4]
        rank = jnp.sum(a * (r_intra + base), axis=1, keepdims=True)
        rank_ref[p, pl.ds(m * MR, MR), :] = rank
        cnt_ref[0:1, 0:4] = base + jnp.sum(a, axis=0, keepdims=True)

    @pl.when(p == 2)
    def _():
        cnt = cnt_ref[0:1, 0:4]
        pad = jnp.ceil(cnt / MB) * MB
        b1 = pad[0:1, 0:1]
        b2 = b1 + pad[0:1, 1:2]
        b3 = b2 + pad[0:1, 2:3]
        for k in range(2):
            a = a_ref[k, pl.ds(m * MR, MR), :]
            r = rank_ref[k, pl.ds(m * MR, MR), :]
            posk = (r + a[:, 1:2] * b1 + a[:, 2:3] * b2 + a[:, 3:4] * b3)
            pos_ref[pl.ds(m * MR, MR), k:k + 1] = posk.astype(I32)

        @pl.when(m == 0)
        def _():
            iob = jax.lax.broadcasted_iota(I32, (8, 128), 1).astype(F32) * MB
            grp = ((iob >= b1).astype(I32) + (iob >= b2).astype(I32)
                   + (iob >= b3).astype(I32))
            grp_ref[...] = grp


def _route(hs, gate_w):
    return pl.pallas_call(
        _route_kernel,
        grid=(3, NMR),
        in_specs=[
            pl.BlockSpec((MR, H), lambda p, m: (m, 0)),
            pl.BlockSpec((E, H), lambda p, m: (0, 0)),
        ],
        out_specs=[
            pl.BlockSpec((2, T, 128), lambda p, m: (0, 0, 0)),
            pl.BlockSpec((T, 8), lambda p, m: (0, 0)),
            pl.BlockSpec((8, 128), lambda p, m: (0, 0)),
        ],
        out_shape=[
            jax.ShapeDtypeStruct((2, T, 128), F32),
            jax.ShapeDtypeStruct((T, 8), I32),
            jax.ShapeDtypeStruct((8, 128), I32),
        ],
        scratch_shapes=[
            pltpu.VMEM((8, 128), F32),
            pltpu.VMEM((2, T, GS), F32),
            pltpu.VMEM((2, T, 1), F32),
        ],
        compiler_params=pltpu.CompilerParams(vmem_limit_bytes=VMEM_LIMIT),
    )(hs, gate_w)


def _dispatch_sc(hs, meta, pos_c):
    mesh = plsc.VectorSubcoreMesh(core_axis_name="c", subcore_axis_name="s")

    @functools.partial(
        pl.kernel, mesh=mesh,
        out_type=[jax.ShapeDtypeStruct((NPAD, H), F32),
                  jax.ShapeDtypeStruct((NPAD, 128), F32)],
        scratch_types=[
            pltpu.VMEM((4, CH), I32),
            pltpu.VMEM((CH, H), F32),
            pltpu.VMEM((2, BPW, 128), F32),
            pltpu.SemaphoreType.DMA,
        ])
    def disp(hs_hbm, meta_hbm, pos_hbm, xg_hbm, ms_hbm,
             idx_v, rows_v, mrow_v, sem):
        wid = jax.lax.axis_index("s") * NC + jax.lax.axis_index("c")
        base = wid * BPW
        for k in range(2):
            for h in range(2):
                pltpu.sync_copy(pos_hbm.at[k, pl.ds(base + h * CH, CH)],
                                idx_v.at[2 * k + h])
        pltpu.sync_copy(meta_hbm.at[0, pl.ds(base, BPW)], mrow_v.at[0])
        pltpu.sync_copy(meta_hbm.at[1, pl.ds(base, BPW)], mrow_v.at[1])
        cps = []
        for k in range(2):
            for h in range(2):
                cps.append(pltpu.async_copy(
                    mrow_v.at[k, pl.ds(h * CH, CH)],
                    ms_hbm.at[idx_v.at[2 * k + h]], sem))
        for c in cps:
            c.wait()
        for h in range(2):
            pltpu.sync_copy(hs_hbm.at[pl.ds(base + h * CH, CH)], rows_v)
            c0 = pltpu.async_copy(rows_v, xg_hbm.at[idx_v.at[h]], sem)
            c1 = pltpu.async_copy(rows_v, xg_hbm.at[idx_v.at[2 + h]], sem)
            c0.wait()
            c1.wait()

    return disp(hs, meta, pos_c)


def _moe_kernel(grp_ref, xg_ref, meta_ref, wg_ref, wu_ref, wd_ref, wm_ref,
                og_ref):
    fsm = meta_ref[:, 0:4]
    scal = meta_ref[:, 4:5]
    delta = _dot_t(fsm.astype(BF), wm_ref[0].astype(BF))
    xb = (xg_ref[...] + delta).astype(BF)
    gate = _dot_t(xb, wg_ref[0])
    up = _dot_t(xb, wu_ref[0])
    hb = (gate * jax.nn.sigmoid(gate) * up).astype(BF)
    og = _dot_t(hb, wd_ref[0])
    og_ref[...] = scal * og


def _moe(grp, xg, meta_s, wg, wu, wd, Wm):
    grid_spec = pltpu.PrefetchScalarGridSpec(
        num_scalar_prefetch=1,
        grid=(NB,),
        in_specs=[
            pl.BlockSpec((MB, H), lambda b, g: (b, 0)),
            pl.BlockSpec((MB, 128), lambda b, g: (b, 0)),
            pl.BlockSpec((1, I, H), lambda b, g: (g[b], 0, 0)),
            pl.BlockSpec((1, I, H), lambda b, g: (g[b], 0, 0)),
            pl.BlockSpec((1, H, I), lambda b, g: (g[b], 0, 0)),
            pl.BlockSpec((1, H, GS), lambda b, g: (g[b], 0, 0)),
        ],
        out_specs=pl.BlockSpec((MB, H), lambda b, g: (b, 0)),
    )
    return pl.pallas_call(
        _moe_kernel,
        grid_spec=grid_spec,
        out_shape=jax.ShapeDtypeStruct((NPAD, H), F32),
        compiler_params=pltpu.CompilerParams(vmem_limit_bytes=VMEM_LIMIT),
    )(grp, xg, meta_s, wg, wu, wd, Wm)


def _combine_sc(og, pos_c):
    mesh = plsc.VectorSubcoreMesh(core_axis_name="c", subcore_axis_name="s")

    @functools.partial(
        pl.kernel, mesh=mesh,
        out_type=jax.ShapeDtypeStruct((2, T, H), F32),
        scratch_types=[
            pltpu.VMEM((4, CH), I32),
            pltpu.VMEM((CH, H), F32),
            pltpu.SemaphoreType.DMA,
        ])
    def comb(og_hbm, pos_hbm, y_hbm, idx_v, rows_v, sem):
        wid = jax.lax.axis_index("s") * NC + jax.lax.axis_index("c")
        base = wid * BPW
        for k in range(2):
            for h in range(2):
                pltpu.sync_copy(pos_hbm.at[k, pl.ds(base + h * CH, CH)],
                                idx_v.at[2 * k + h])
        for k in range(2):
            for h in range(2):
                pltpu.async_copy(og_hbm.at[idx_v.at[2 * k + h]], rows_v,
                                 sem).wait()
                pltpu.sync_copy(
                    rows_v, y_hbm.at[k, pl.ds(base + h * CH, CH)])

    return comb(og, pos_c)


def _shared_kernel(hs_ref, wsg_ref, wsu_ref, wsd_ref, out_ref):
    xb = hs_ref[...].astype(BF)
    gate = _dot_t(xb, wsg_ref[...])
    up = _dot_t(xb, wsu_ref[...])
    hb = (gate * jax.nn.sigmoid(gate) * up).astype(BF)
    out_ref[...] = _dot_t(hb, wsd_ref[...])


def _shared(hs, wsg, wsu, wsd):
    return pl.pallas_call(
        _shared_kernel,
        grid=(T // MS,),
        in_specs=[
            pl.BlockSpec((MS, H), lambda m: (m, 0)),
            pl.BlockSpec((SH_I, H), lambda m: (0, 0)),
            pl.BlockSpec((SH_I, H), lambda m: (0, 0)),
            pl.BlockSpec((H, SH_I), lambda m: (0, 0)),
        ],
        out_specs=pl.BlockSpec((MS, H), lambda m: (m, 0)),
        out_shape=jax.ShapeDtypeStruct((T, H), F32),
        compiler_params=pltpu.CompilerParams(vmem_limit_bytes=VMEM_LIMIT),
    )(hs, wsg, wsu, wsd)


def _final_kernel(sh_ref, y_ref, out_ref):
    out_ref[...] = y_ref[0] + y_ref[1] + sh_ref[...]


def _final(sh, y):
    return pl.pallas_call(
        _final_kernel,
        grid=(T // MS,),
        in_specs=[
            pl.BlockSpec((MS, H), lambda m: (m, 0)),
            pl.BlockSpec((2, MS, H), lambda m: (0, m, 0)),
        ],
        out_specs=pl.BlockSpec((MS, H), lambda m: (m, 0)),
        out_shape=jax.ShapeDtypeStruct((T, H), F32),
        compiler_params=pltpu.CompilerParams(vmem_limit_bytes=VMEM_LIMIT),
    )(sh, y)


def kernel(hidden_states, gate_w, Wg, Wu, Wd, Wm, Wsg, Wsu, Wsd):
    hs = hidden_states.reshape(T, H)
    wg = Wg.astype(BF)
    wu = Wu.astype(BF)
    wd = Wd.astype(BF)
    wsg = Wsg.astype(BF)
    wsu = Wsu.astype(BF)
    wsd = Wsd.astype(BF)

    meta, pos, grp8 = _route(hs, gate_w)
    shared = _shared(hs, wsg, wsu, wsd)
    pos_c = pos.T[:2]
    grp = grp8[0, :NB]

    xg, meta_s = _dispatch_sc(hs, meta, pos_c)

    og = _moe(grp, xg, meta_s, wg, wu, wd, Wm)

    y = _combine_sc(og, pos_c)

    out = _final(shared, y)
    return out.reshape(1, S, H)
